# Initial kernel scaffold; baseline (speedup 1.0000x reference)
import jax
import jax.numpy as jnp
from jax.experimental import pallas as pl


def kernel(x, edge_index, W1, b1, W2, b2):
    raise NotImplementedError("write your pallas kernel here")


# speedup vs baseline: 5.7629x; 5.7629x over previous
import functools

import jax
import jax.numpy as jnp
from jax import lax
from jax.experimental import pallas as pl
from jax.experimental.pallas import tpu as pltpu
from jax.experimental.pallas import tpu_sc as plsc

N = 10000
E = 160000
DIN = 162
DH = 512
DHALF = 88
ONES_COL = 74
CH = 128
NCHUNK = E // CH
NS = 16
RB = 80
NRB = N // RB
BLK = 1000


def _seg_body(tab0, tab1, uu, vv, o0, o1, zb, ub, vb, rows, sem, acc):
    c = lax.axis_index("c")
    s = lax.axis_index("s")

    def run(tab, out):
        nrb = jnp.where(s < NRB - (NRB // NS) * NS, NRB // NS + 1, NRB // NS)

        def initb(k, carry):
            r0 = pl.multiple_of((s + k * NS) * RB, 8)
            pltpu.sync_copy(tab.at[pl.ds(r0, RB)], zb)
            pltpu.sync_copy(zb, acc.at[pl.ds(r0, RB)])
            return carry

        lax.fori_loop(0, nrb, initb, 0)
        plsc.subcore_barrier()
        nch = jnp.where(s < NCHUNK - (NCHUNK // NS) * NS, NCHUNK // NS + 1,
                        NCHUNK // NS)

        def body(i, carry):
            base = pl.multiple_of((s + i * NS) * CH, 8)
            pltpu.sync_copy(uu.at[pl.ds(base, CH)], ub)
            pltpu.sync_copy(vv.at[pl.ds(base, CH)], vb)
            pltpu.async_copy(tab.at[ub], rows, sem).wait()
            pltpu.sync_copy(rows, acc.at[vb], add=True)
            return carry

        lax.fori_loop(0, nch, body, 0)
        plsc.subcore_barrier()

        def outb(k, carry):
            r0 = pl.multiple_of((s + k * NS) * RB, 8)
            pltpu.sync_copy(acc.at[pl.ds(r0, RB)], zb)
            pltpu.sync_copy(zb, out.at[pl.ds(r0, RB)])
            return carry

        lax.fori_loop(0, nrb, outb, 0)

    @pl.when(c == 0)
    def _():
        run(tab0, o0)

    @pl.when(c == 1)
    def _():
        run(tab1, o1)


_seg = functools.partial(
    pl.kernel,
    out_type=(jax.ShapeDtypeStruct((N, DHALF), jnp.float32),
              jax.ShapeDtypeStruct((N, DHALF), jnp.float32)),
    mesh=plsc.VectorSubcoreMesh(core_axis_name="c", subcore_axis_name="s"),
    scratch_types=[
        pltpu.VMEM((RB, DHALF), jnp.float32),
        pltpu.VMEM((CH,), jnp.int32),
        pltpu.VMEM((CH,), jnp.int32),
        pltpu.VMEM((CH, DHALF), jnp.float32),
        pltpu.SemaphoreType.DMA,
        pltpu.VMEM_SHARED((N, DHALF), jnp.float32),
    ],
    compiler_params=pltpu.CompilerParams(use_tc_tiling_on_sc=False),
)(_seg_body)


def _mm_body(za1, wa0, wa1, w1a, w1b, b1, w2, b2, out):
    t = jnp.dot(wa0[...], w1a[...], preferred_element_type=jnp.float32)
    t += jnp.dot(wa1[...], w1b[...], preferred_element_type=jnp.float32)
    t += za1[:, ONES_COL:ONES_COL + 1] * b1[...]
    out[...] = jnp.dot(t, w2[...], preferred_element_type=jnp.float32) + b2[...]


_mm = pl.pallas_call(
    _mm_body,
    grid=(N // BLK,),
    in_specs=[
        pl.BlockSpec((BLK, DHALF), lambda i: (i, 0)),
        pl.BlockSpec((BLK, DHALF), lambda i: (i, 0)),
        pl.BlockSpec((BLK, DHALF), lambda i: (i, 0)),
        pl.BlockSpec((DHALF, DH), lambda i: (0, 0)),
        pl.BlockSpec((DHALF, DH), lambda i: (0, 0)),
        pl.BlockSpec((1, DH), lambda i: (0, 0)),
        pl.BlockSpec((DH, DH), lambda i: (0, 0)),
        pl.BlockSpec((1, DH), lambda i: (0, 0)),
    ],
    out_specs=pl.BlockSpec((BLK, DH), lambda i: (i, 0)),
    out_shape=jax.ShapeDtypeStruct((N, DH), jnp.float32),
)


def kernel(x, edge_index, W1, b1, W2, b2):
    u = edge_index[0].astype(jnp.int32)
    v = edge_index[1].astype(jnp.int32)
    xa0 = x[:, :DHALF]
    xa1 = jnp.concatenate(
        [x[:, DHALF:], jnp.ones((N, 1), jnp.float32),
         jnp.zeros((N, DHALF - (DIN - DHALF) - 1), jnp.float32)], axis=1)
    za0, za1 = _seg(xa0, xa1, u, v)
    wa0, wa1 = _seg(za0, za1, u, v)
    w1a = W1[:DHALF]
    w1b = jnp.pad(W1[DHALF:], ((0, DHALF - (DIN - DHALF)), (0, 0)))
    return _mm(za1, wa0, wa1, w1a, w1b, b1.reshape(1, DH), W2,
               b2.reshape(1, DH))


# speedup vs baseline: 5.9542x; 1.0332x over previous
import functools

import jax
import jax.numpy as jnp
from jax import lax
from jax.experimental import pallas as pl
from jax.experimental.pallas import tpu as pltpu
from jax.experimental.pallas import tpu_sc as plsc

N = 10000
NP = 10240
E = 160000
EP = 163840
DIN = 162
DH = 512
DHALF = 88
ONES_COL = 74
CH = 128
NCHUNK = EP // CH
NS = 16
CPT = NCHUNK // NS
RPT = NP // NS
NSLOT = 4
NTURN = CPT // NSLOT
BLK = 1000


def _seg_body(tab0, tab1, uu, vv, o0, o1, ub, vb, rows, gsem, ssem, acc):
    c = lax.axis_index("c")
    s = lax.axis_index("s")

    def rslot(b):
        return rows.at[pl.ds(b * CH, CH)]

    def run(tab, out):
        c0 = pl.multiple_of(s * CPT, 8)
        pltpu.sync_copy(uu.at[pl.ds(c0, CPT)], ub)
        pltpu.sync_copy(vv.at[pl.ds(c0, CPT)], vb)
        r0 = pl.multiple_of(s * RPT, 8)
        pltpu.sync_copy(tab.at[pl.ds(r0, NSLOT * CH)], rows)
        pltpu.sync_copy(rows, acc.at[pl.ds(r0, NSLOT * CH)])
        r1 = pl.multiple_of(s * RPT + NSLOT * CH, 8)
        pltpu.sync_copy(tab.at[pl.ds(r1, CH)], rslot(0))
        pltpu.sync_copy(rslot(0), acc.at[pl.ds(r1, CH)])
        plsc.subcore_barrier()

        def g_start(k, b):
            pltpu.async_copy(tab.at[ub.at[k]], rslot(b), gsem.at[b])

        def g_wait(k, b):
            pltpu.make_async_copy(tab.at[ub.at[k]], rslot(b),
                                  gsem.at[b]).wait()

        def s_start(k, b):
            pltpu.async_copy(rslot(b), acc.at[vb.at[k]], ssem.at[b],
                             add=True)

        def s_wait(k, b):
            pltpu.make_async_copy(rslot(b), acc.at[vb.at[k]],
                                  ssem.at[b]).wait()

        g_start(0, 0)
        g_start(1, 1)

        def turn(j, carry):
            for b in range(NSLOT):
                k = j * NSLOT + b
                g_wait(k, b)
                s_start(k, b)

                @pl.when(k >= 2)
                def _():
                    s_wait(k - 2, (b + 2) % NSLOT)

                @pl.when(k + 2 < CPT)
                def _():
                    g_start(k + 2, (b + 2) % NSLOT)
            return carry

        lax.fori_loop(0, NTURN, turn, 0)
        s_wait(CPT - 2, (CPT - 2) % NSLOT)
        s_wait(CPT - 1, (CPT - 1) % NSLOT)
        plsc.subcore_barrier()

        pltpu.sync_copy(acc.at[pl.ds(r0, NSLOT * CH)], rows)
        pltpu.sync_copy(rows, out.at[pl.ds(r0, NSLOT * CH)])
        pltpu.sync_copy(acc.at[pl.ds(r1, CH)], rslot(0))
        pltpu.sync_copy(rslot(0), out.at[pl.ds(r1, CH)])

    @pl.when(c == 0)
    def _():
        run(tab0, o0)

    @pl.when(c == 1)
    def _():
        run(tab1, o1)


_seg = functools.partial(
    pl.kernel,
    out_type=(jax.ShapeDtypeStruct((NP, DHALF), jnp.float32),
              jax.ShapeDtypeStruct((NP, DHALF), jnp.float32)),
    mesh=plsc.VectorSubcoreMesh(core_axis_name="c", subcore_axis_name="s"),
    scratch_types=[
        pltpu.VMEM((CPT, CH), jnp.int32),
        pltpu.VMEM((CPT, CH), jnp.int32),
        pltpu.VMEM((NSLOT * CH, DHALF), jnp.float32),
        pltpu.SemaphoreType.DMA((NSLOT,)),
        pltpu.SemaphoreType.DMA((NSLOT,)),
        pltpu.VMEM_SHARED((NP, DHALF), jnp.float32),
    ],
    compiler_params=pltpu.CompilerParams(use_tc_tiling_on_sc=False),
)(_seg_body)


def _mm_body(za1, wa0, wa1, w1a, w1b, b1, w2, b2, out):
    t = jnp.dot(wa0[...], w1a[...], preferred_element_type=jnp.float32)
    t += jnp.dot(wa1[...], w1b[...], preferred_element_type=jnp.float32)
    t += za1[:, ONES_COL:ONES_COL + 1] * b1[...]
    out[...] = jnp.dot(t, w2[...], preferred_element_type=jnp.float32) + b2[...]


_mm = pl.pallas_call(
    _mm_body,
    grid=(N // BLK,),
    in_specs=[
        pl.BlockSpec((BLK, DHALF), lambda i: (i, 0)),
        pl.BlockSpec((BLK, DHALF), lambda i: (i, 0)),
        pl.BlockSpec((BLK, DHALF), lambda i: (i, 0)),
        pl.BlockSpec((DHALF, DH), lambda i: (0, 0)),
        pl.BlockSpec((DHALF, DH), lambda i: (0, 0)),
        pl.BlockSpec((1, DH), lambda i: (0, 0)),
        pl.BlockSpec((DH, DH), lambda i: (0, 0)),
        pl.BlockSpec((1, DH), lambda i: (0, 0)),
    ],
    out_specs=pl.BlockSpec((BLK, DH), lambda i: (i, 0)),
    out_shape=jax.ShapeDtypeStruct((N, DH), jnp.float32),
)


def kernel(x, edge_index, W1, b1, W2, b2):
    pad = jnp.full((EP - E,), N, jnp.int32)
    u = jnp.concatenate([edge_index[0].astype(jnp.int32), pad]).reshape(
        NCHUNK, CH)
    v = jnp.concatenate([edge_index[1].astype(jnp.int32), pad]).reshape(
        NCHUNK, CH)
    xa0 = jnp.pad(x[:, :DHALF], ((0, NP - N), (0, 0)))
    xa1 = jnp.pad(
        jnp.concatenate([x[:, DHALF:], jnp.ones((N, 1), jnp.float32)],
                        axis=1),
        ((0, NP - N), (0, DHALF - (DIN - DHALF) - 1)))
    za0, za1 = _seg(xa0, xa1, u, v)
    wa0, wa1 = _seg(za0, za1, u, v)
    w1a = W1[:DHALF]
    w1b = jnp.pad(W1[DHALF:], ((0, DHALF - (DIN - DHALF)), (0, 0)))
    return _mm(za1[:N], wa0[:N], wa1[:N], w1a, w1b, b1.reshape(1, DH), W2,
               b2.reshape(1, DH))
